# baseline (speedup 1.0000x reference)
import functools

import numpy as np
import jax
import jax.numpy as jnp
from jax import lax
from jax.experimental import pallas as pl
from jax.experimental.pallas import tpu as pltpu

_GROUP = 128


def _gem_body(x_ref, w_ref, o_ref, *, eps, inv_cols, inv_p):
    x = jnp.maximum(x_ref[...], eps)
    c = (x * x * x).astype(jnp.bfloat16)
    s = lax.dot_general(
        c, w_ref[...],
        dimension_numbers=(((1,), (0,)), ((), ())),
        preferred_element_type=jnp.float32)
    m = s * inv_cols
    o_ref[...] = jnp.exp(jnp.log(m) * inv_p)


@functools.partial(jax.jit, static_argnames=("p", "eps"))
def _gem_pool(x, p=3.0, eps=1e-6):
    N, C, H, W = x.shape
    rows, cols = N * C, H * W
    G = _GROUP
    assert rows % G == 0, "row count must pack into 128-row super-rows"
    S = rows // G
    L = G * cols

    x2 = x.reshape(S, L)

    w = jnp.asarray(np.repeat(np.eye(G, dtype=np.float32), cols, axis=0),
                    dtype=jnp.bfloat16)

    tile_R = min(S, 128)
    grid = (pl.cdiv(S, tile_R),)

    body = functools.partial(
        _gem_body, eps=float(eps), inv_cols=1.0 / float(cols),
        inv_p=1.0 / float(p))

    y = pl.pallas_call(
        body,
        out_shape=jax.ShapeDtypeStruct((S, G), jnp.float32),
        grid=grid,
        in_specs=[
            pl.BlockSpec((tile_R, L), lambda i: (i, 0)),
            pl.BlockSpec((L, G), lambda i: (0, 0)),
        ],
        out_specs=pl.BlockSpec((tile_R, G), lambda i: (i, 0)),
        compiler_params=pltpu.CompilerParams(
            dimension_semantics=("parallel",)),
    )(x2, w)

    return y.reshape(rows).astype(x.dtype).reshape(N, C, 1, 1)


def kernel(x):
    return _gem_pool(x, 3.0, eps=1e-6)
